# Initial kernel scaffold; baseline (speedup 1.0000x reference)
import jax
import jax.numpy as jnp
from jax.experimental import pallas as pl


def kernel(dense_x, dense_padding_mask):
    raise NotImplementedError("write your pallas kernel here")


# speedup vs baseline: 1.0154x; 1.0154x over previous
import jax
import jax.numpy as jnp
from jax.experimental import pallas as pl


def _body(x_ref, m_ref, o_ref, mo_ref):
    o_ref[:] = jnp.sum(x_ref[:], axis=1) * 0.25
    mo_ref[:] = jnp.min(m_ref[:], axis=1, keepdims=True)


def kernel(dense_x, dense_padding_mask):
    bsz, tsz, fsz = dense_x.shape
    gs = 4
    tn = tsz // gs
    rows = bsz * tn

    x4 = dense_x.reshape(rows, gs, fsz)
    m4 = dense_padding_mask.reshape(rows, gs).astype(jnp.int32)

    R = 256
    grid = rows // R

    out, mout = pl.pallas_call(
        _body,
        grid=(grid,),
        in_specs=[
            pl.BlockSpec((R, gs, fsz), lambda i: (i, 0, 0)),
            pl.BlockSpec((R, gs), lambda i: (i, 0)),
        ],
        out_specs=[
            pl.BlockSpec((R, fsz), lambda i: (i, 0)),
            pl.BlockSpec((R, 1), lambda i: (i, 0)),
        ],
        out_shape=[
            jax.ShapeDtypeStruct((rows, fsz), jnp.float32),
            jax.ShapeDtypeStruct((rows, 1), jnp.int32),
        ],
    )(x4, m4)

    return (
        out.reshape(bsz, tn, fsz),
        mout.reshape(bsz, tn).astype(jnp.bool_),
    )


# speedup vs baseline: 1.2087x; 1.1903x over previous
import jax
import jax.numpy as jnp
from jax.experimental import pallas as pl


def _make_body(fsz):
    def _body(x_ref, m_ref, o_ref, mo_ref):
        x = x_ref[:]
        acc = x[:, 0:fsz] + x[:, fsz : 2 * fsz]
        acc = acc + x[:, 2 * fsz : 3 * fsz]
        acc = acc + x[:, 3 * fsz : 4 * fsz]
        o_ref[:] = acc * 0.25
        mo_ref[:] = jnp.min(m_ref[:], axis=1, keepdims=True)

    return _body


def kernel(dense_x, dense_padding_mask):
    bsz, tsz, fsz = dense_x.shape
    gs = 4
    tn = tsz // gs
    rows = bsz * tn

    x2 = dense_x.reshape(rows, gs * fsz)
    m4 = dense_padding_mask.reshape(rows, gs).astype(jnp.int32)

    R = 512
    grid = rows // R

    out, mout = pl.pallas_call(
        _make_body(fsz),
        grid=(grid,),
        in_specs=[
            pl.BlockSpec((R, gs * fsz), lambda i: (i, 0)),
            pl.BlockSpec((R, gs), lambda i: (i, 0)),
        ],
        out_specs=[
            pl.BlockSpec((R, fsz), lambda i: (i, 0)),
            pl.BlockSpec((R, 1), lambda i: (i, 0)),
        ],
        out_shape=[
            jax.ShapeDtypeStruct((rows, fsz), jnp.float32),
            jax.ShapeDtypeStruct((rows, 1), jnp.int32),
        ],
    )(x2, m4)

    return (
        out.reshape(bsz, tn, fsz),
        mout.reshape(bsz, tn).astype(jnp.bool_),
    )
